# Initial kernel scaffold; baseline (speedup 1.0000x reference)
import jax
import jax.numpy as jnp
from jax.experimental import pallas as pl


def kernel(atomic_numbers, pos, edge_index, centers, widths, node_emb, layers, readout, atomic_e):
    raise NotImplementedError("write your pallas kernel here")


# speedup vs baseline: 1.8405x; 1.8405x over previous
import functools
import math

import jax
import jax.numpy as jnp
from jax import lax
from jax.experimental import pallas as pl
from jax.experimental.pallas import tpu as pltpu
from jax.experimental.pallas import tpu_sc as plsc

CUTOFF = 5.0
F32 = jnp.float32
I32 = jnp.int32
CHUNK = 128
EBLK = 3200
NBLK = 400


def _silu(x):
    return x * jax.nn.sigmoid(x)


def _dotg(a, b):
    return lax.dot_general(a, b, (((1,), (1,)), ((), ())),
                           preferred_element_type=F32)


def _edge_d2_body(posx, posy, posz, row2d, col2d, out,
                  idx_r, idx_c, bxr, byr, bzr, bxc, byc, bzc, d2b, sem):
    c = lax.axis_index("c")
    s = lax.axis_index("s")
    wid = s * 2 + c
    nchunks = row2d.shape[0]
    start = (wid * nchunks) // 32
    end = ((wid + 1) * nchunks) // 32

    def body(ch, carry):
        pltpu.sync_copy(row2d.at[ch], idx_r)
        pltpu.sync_copy(col2d.at[ch], idx_c)
        cps = [
            pltpu.async_copy(posx.at[idx_r], bxr, sem),
            pltpu.async_copy(posy.at[idx_r], byr, sem),
            pltpu.async_copy(posz.at[idx_r], bzr, sem),
            pltpu.async_copy(posx.at[idx_c], bxc, sem),
            pltpu.async_copy(posy.at[idx_c], byc, sem),
            pltpu.async_copy(posz.at[idx_c], bzc, sem),
        ]
        for cp in cps:
            cp.wait()

        def grp(g, carry2):
            sl = pl.ds(g * 16, 16)
            dx = bxc[sl] - bxr[sl]
            dy = byc[sl] - byr[sl]
            dz = bzc[sl] - bzr[sl]
            d2b[sl] = dx * dx + dy * dy + dz * dz
            return carry2

        lax.fori_loop(0, CHUNK // 16, grp, 0)
        pltpu.sync_copy(d2b, out.at[pl.ds(ch * CHUNK, CHUNK)])
        return carry

    lax.fori_loop(start, end, body, 0)


def _msg_body(row2d, col2d, wlo, whi, flo, fhi, outlo, outhi,
              idx_r, idx_c, gbuf, wbuf, zbuf, acc, semg):
    c = lax.axis_index("c")
    s = lax.axis_index("s")
    nchunks = row2d.shape[0]
    n_nodes = flo.shape[0]
    zrows = zbuf.shape[0]
    n_zchunks = n_nodes // zrows
    zstart = (s * n_zchunks) // 16
    zend = ((s + 1) * n_zchunks) // 16

    zv = jnp.zeros((16,), F32)

    def zb(i, carry):
        for j in range(2):
            zbuf[i, pl.ds(j * 16, 16)] = zv
        return carry

    lax.fori_loop(0, zrows, zb, 0)

    def zslice(k, carry):
        pltpu.sync_copy(zbuf, acc.at[pl.ds(k * zrows, zrows)])
        return carry

    lax.fori_loop(zstart, zend, zslice, 0)
    plsc.subcore_barrier()

    start = (s * nchunks) // 16
    end = ((s + 1) * nchunks) // 16

    def run(fsrc, wsrc, dst):
        def body(ch, carry):
            pltpu.sync_copy(row2d.at[ch], idx_r)
            pltpu.sync_copy(col2d.at[ch], idx_c)
            pltpu.async_copy(fsrc.at[idx_c], gbuf, semg).wait()
            pltpu.sync_copy(wsrc.at[pl.ds(ch * CHUNK, CHUNK)], wbuf)

            def mul(i, carry2):
                for u in range(8):
                    r = i * 8 + u
                    for j in range(2):
                        sl = pl.ds(j * 16, 16)
                        gbuf[r, sl] = gbuf[r, sl] * wbuf[r, sl]
                return carry2

            lax.fori_loop(0, CHUNK // 8, mul, 0)
            pltpu.sync_copy(gbuf, acc.at[idx_r], add=True)
            return carry

        lax.fori_loop(start, end, body, 0)
        plsc.subcore_barrier()

        def wb(k, carry):
            pltpu.sync_copy(acc.at[pl.ds(k * zrows, zrows)],
                            dst.at[pl.ds(k * zrows, zrows)])
            return carry

        lax.fori_loop(zstart, zend, wb, 0)

    @pl.when(c == 0)
    def _():
        run(flo, wlo, outlo)

    @pl.when(c == 1)
    def _():
        run(fhi, whi, outhi)


def _edge_w_body(d2_ref, cen_ref, wid_ref, w1_ref, b1_ref, w2_ref, b2_ref,
                 *out_refs):
    d2 = jnp.broadcast_to(d2_ref[...], (d2_ref.shape[0], 8))
    ln = jnp.sqrt(d2)
    xx = ln * (1.0 / CUTOFF)
    cut = 0.5 * (jnp.cos(xx * math.pi) + 1.0)
    cut = cut * (ln < CUTOFF).astype(F32)
    wc = jnp.clip(wid_ref[...], 0.1, None)
    diff = (ln - cen_ref[...]) / wc
    basis = jnp.exp(-0.5 * diff * diff) * cut
    for l in range(3):
        h1 = _dotg(basis, w1_ref[l]) + b1_ref[l][None, :]
        h1 = _silu(h1)
        w = _dotg(h1, w2_ref[l]) + b2_ref[l][None, :]
        out_refs[2 * l][...] = w[:, :32]
        out_refs[2 * l + 1][...] = w[:, 32:]


def _embed_body(z_ref, emb_ref, olo_ref, ohi_ref):
    z = z_ref[...]
    ids = lax.broadcasted_iota(I32, (z.shape[0], 128), 1)
    oh = (ids == z).astype(F32)
    f = lax.dot_general(oh, emb_ref[...], (((1,), (0,)), ((), ())),
                        preferred_element_type=F32)
    olo_ref[...] = f[:, :32]
    ohi_ref[...] = f[:, 32:]


def _node_body(flo_ref, fhi_ref, alo_ref, ahi_ref,
               siw_ref, sib_ref, cpw_ref, cpb_ref,
               uw1_ref, ub1_ref, uw2_ref, ub2_ref, lng_ref, lnb_ref,
               olo_ref, ohi_ref):
    feats = jnp.concatenate([flo_ref[...], fhi_ref[...]], axis=1)
    agg = jnp.concatenate([alo_ref[...], ahi_ref[...]], axis=1)
    self_out = _dotg(feats, siw_ref[...]) + sib_ref[...]
    combined = jnp.concatenate([self_out, agg], axis=1)
    conv = _dotg(combined, cpw_ref[...]) + cpb_ref[...]
    u1 = _silu(_dotg(conv, uw1_ref[...]) + ub1_ref[...])
    upd = _dotg(u1, uw2_ref[...]) + ub2_ref[...]
    h = feats + upd
    mu = jnp.mean(h, axis=1, keepdims=True)
    var = jnp.mean((h - mu) ** 2, axis=1, keepdims=True)
    out = (h - mu) / jnp.sqrt(var + 1e-5) * lng_ref[...] + lnb_ref[...]
    olo_ref[...] = out[:, :32]
    ohi_ref[...] = out[:, 32:]


def _readout_body(flo_ref, fhi_ref, z_ref,
                  w1_ref, b1_ref, w2_ref, b2_ref, w3_ref, b3_ref, ae_ref,
                  out_ref):
    feats = jnp.concatenate([flo_ref[...], fhi_ref[...]], axis=1)
    e = _silu(_dotg(feats, w1_ref[...]) + b1_ref[...])
    e = _silu(_dotg(e, w2_ref[...]) + b2_ref[...])
    z = z_ref[...]
    ids = lax.broadcasted_iota(I32, (z.shape[0], 128), 1)
    oh = (ids == z).astype(F32)
    si = (jnp.sum(e * w3_ref[...])
          + z.shape[0] * jnp.sum(b3_ref[...])
          + jnp.sum(oh * ae_ref[...]))

    @pl.when(pl.program_id(0) == 0)
    def _():
        out_ref[...] = jnp.zeros_like(out_ref)

    out_ref[...] = out_ref[...] + si


def kernel(atomic_numbers, pos, edge_index, centers, widths, node_emb,
           layers, readout, atomic_e):
    n_nodes = pos.shape[0]
    n_edges = edge_index.shape[1]
    hidden = node_emb.shape[1]
    half = hidden // 2
    nchunks = n_edges // CHUNK

    row2d = edge_index[0].astype(I32).reshape(nchunks, CHUNK)
    col2d = edge_index[1].astype(I32).reshape(nchunks, CHUNK)
    posx = pos[:, 0].astype(F32)
    posy = pos[:, 1].astype(F32)
    posz = pos[:, 2].astype(F32)
    z2d = atomic_numbers.astype(I32).reshape(n_nodes, 1)
    emb_pad = jnp.pad(node_emb.astype(F32), ((0, 128 - node_emb.shape[0]), (0, 0)))
    ae_pad = jnp.pad(atomic_e.astype(F32), ((0, 128 - atomic_e.shape[0]), (0, 0))).reshape(1, 128)
    cen2d = centers.astype(F32).reshape(1, -1)
    wid2d = widths.astype(F32).reshape(1, -1)

    mesh = plsc.VectorSubcoreMesh(core_axis_name="c", subcore_axis_name="s",
                                  num_cores=2, num_subcores=16)

    d2 = pl.kernel(
        _edge_d2_body,
        out_type=jax.ShapeDtypeStruct((n_edges,), F32),
        mesh=mesh,
        compiler_params=pltpu.CompilerParams(use_tc_tiling_on_sc=False),
        scratch_types=[
            pltpu.VMEM((CHUNK,), I32),
            pltpu.VMEM((CHUNK,), I32),
        ] + [pltpu.VMEM((CHUNK,), F32)] * 7 + [
            pltpu.SemaphoreType.DMA,
        ],
    )(posx, posy, posz, row2d, col2d)
    d2col = d2.reshape(n_edges, 1)

    w1s = jnp.stack([p['rn_W1'].astype(F32) for p in layers])
    b1s = jnp.stack([p['rn_b1'].astype(F32) for p in layers])
    w2s = jnp.stack([p['rn_W2'].astype(F32) for p in layers])
    b2s = jnp.stack([p['rn_b2'].astype(F32) for p in layers])

    egrid = n_edges // EBLK
    full = lambda shp: pl.BlockSpec(shp, lambda i: (0,) * len(shp))
    wouts = pl.pallas_call(
        _edge_w_body,
        grid=(egrid,),
        in_specs=[
            pl.BlockSpec((EBLK, 1), lambda i: (i, 0)),
            full((1, 8)), full((1, 8)),
            full((3, hidden, 8)), full((3, hidden)),
            full((3, hidden, hidden)), full((3, hidden)),
        ],
        out_specs=[pl.BlockSpec((EBLK, half), lambda i: (i, 0))] * 6,
        out_shape=[jax.ShapeDtypeStruct((n_edges, half), F32)] * 6,
    )(d2col, cen2d, wid2d, w1s, b1s, w2s, b2s)

    ngrid = n_nodes // NBLK
    flo, fhi = pl.pallas_call(
        _embed_body,
        grid=(ngrid,),
        in_specs=[pl.BlockSpec((NBLK, 1), lambda i: (i, 0)),
                  full((128, hidden))],
        out_specs=[pl.BlockSpec((NBLK, half), lambda i: (i, 0))] * 2,
        out_shape=[jax.ShapeDtypeStruct((n_nodes, half), F32)] * 2,
    )(z2d, emb_pad)

    msg_call = pl.kernel(
        _msg_body,
        out_type=[jax.ShapeDtypeStruct((n_nodes, half), F32)] * 2,
        mesh=mesh,
        compiler_params=pltpu.CompilerParams(use_tc_tiling_on_sc=False),
        scratch_types=[
            pltpu.VMEM((CHUNK,), I32),
            pltpu.VMEM((CHUNK,), I32),
            pltpu.VMEM((CHUNK, half), F32),
            pltpu.VMEM((CHUNK, half), F32),
            pltpu.VMEM((400, half), F32),
            pltpu.VMEM_SHARED((n_nodes, half), F32),
            pltpu.SemaphoreType.DMA,
        ],
    )

    for li, p in enumerate(layers):
        wlo, whi = wouts[2 * li], wouts[2 * li + 1]
        alo, ahi = msg_call(row2d, col2d, wlo, whi, flo, fhi)
        flo, fhi = pl.pallas_call(
            _node_body,
            grid=(ngrid,),
            in_specs=[pl.BlockSpec((NBLK, half), lambda i: (i, 0))] * 4 + [
                full((hidden, hidden)), full((1, hidden)),
                full((hidden, 2 * hidden)), full((1, hidden)),
                full((2 * hidden, hidden)), full((1, 2 * hidden)),
                full((hidden, 2 * hidden)), full((1, hidden)),
                full((1, hidden)), full((1, hidden)),
            ],
            out_specs=[pl.BlockSpec((NBLK, half), lambda i: (i, 0))] * 2,
            out_shape=[jax.ShapeDtypeStruct((n_nodes, half), F32)] * 2,
        )(flo, fhi, alo, ahi,
          p['si_W'].astype(F32), p['si_b'].astype(F32).reshape(1, -1),
          p['cp_W'].astype(F32), p['cp_b'].astype(F32).reshape(1, -1),
          p['u_W1'].astype(F32), p['u_b1'].astype(F32).reshape(1, -1),
          p['u_W2'].astype(F32), p['u_b2'].astype(F32).reshape(1, -1),
          p['ln_g'].astype(F32).reshape(1, -1),
          p['ln_b'].astype(F32).reshape(1, -1))

    etot = pl.pallas_call(
        _readout_body,
        grid=(ngrid,),
        in_specs=[pl.BlockSpec((NBLK, half), lambda i: (i, 0))] * 2 + [
            pl.BlockSpec((NBLK, 1), lambda i: (i, 0)),
            full((hidden, hidden)), full((1, hidden)),
            full((half, hidden)), full((1, half)),
            full((1, half)), full((1, 1)),
            full((1, 128)),
        ],
        out_specs=pl.BlockSpec((1, 1), lambda i: (0, 0)),
        out_shape=jax.ShapeDtypeStruct((1, 1), F32),
    )(flo, fhi, z2d,
      readout['W1'].astype(F32), readout['b1'].astype(F32).reshape(1, -1),
      readout['W2'].astype(F32), readout['b2'].astype(F32).reshape(1, -1),
      readout['W3'].astype(F32), readout['b3'].astype(F32).reshape(1, -1),
      ae_pad)

    return etot[0, 0]
